# baseline (speedup 1.0000x reference)
import functools

import jax
import jax.numpy as jnp
from jax import lax
from jax.experimental import pallas as pl
from jax.experimental.pallas import tpu as pltpu
from jax.experimental.pallas import tpu_sc as plsc

N = 10000
T = 8
F = 128
E = 320000
D = 128
H = 8
DH = 16

NC = 2
NS = 16
EDGES_PER_TILE = E // (NC * NS)
C = 224
CC = 400
TGT = 2048
TPT = TGT // NS
GARB = TGT
SLOTS = TGT + 16
SROWS = SLOTS // NS
CBUF = EDGES_PER_TILE + 2 * C + 16

BLK1 = 2000
BLK3 = 256


def _prep_body(x_ref, w_ref, a_ref, h_ref, as_ref, ad_ref, maxa_ref):
    b = pl.program_id(0)
    hb = jnp.dot(x_ref[...], w_ref[...], preferred_element_type=jnp.float32)
    ab = jnp.dot(hb, a_ref[...], preferred_element_type=jnp.float32)
    h_ref[...] = hb
    as_ref[...] = ab[:, 0:16]
    ad_ref[...] = ab[:, 16:32]
    lane = lax.broadcasted_iota(jnp.int32, (1, 128), 1)
    colmax = jnp.max(ab, axis=0, keepdims=True)
    m = jnp.where(lane < 16, colmax, 0.0)

    @pl.when(b == 0)
    def _():
        maxa_ref[...] = m

    @pl.when(b != 0)
    def _():
        maxa_ref[...] = jnp.maximum(maxa_ref[...], m)


def _prep_call(x2, W_s, A128):
    nblk = (T * N) // BLK1
    return pl.pallas_call(
        _prep_body,
        grid=(nblk,),
        in_specs=[
            pl.BlockSpec((BLK1, F), lambda b: (b, 0)),
            pl.BlockSpec((F, D), lambda b: (0, 0)),
            pl.BlockSpec((D, 128), lambda b: (0, 0)),
        ],
        out_specs=[
            pl.BlockSpec((BLK1, D), lambda b: (b, 0)),
            pl.BlockSpec((BLK1, 16), lambda b: (b, 0)),
            pl.BlockSpec((BLK1, 16), lambda b: (b, 0)),
            pl.BlockSpec((1, 128), lambda b: (0, 0)),
        ],
        out_shape=[
            jax.ShapeDtypeStruct((T * N, D), jnp.float32),
            jax.ShapeDtypeStruct((T * N, 16), jnp.float32),
            jax.ShapeDtypeStruct((T * N, 16), jnp.float32),
            jax.ShapeDtypeStruct((1, 128), jnp.float32),
        ],
    )(x2, W_s, A128)


def _edge_body(src_hbm, dst_hbm, as_hbm, ad_hbm, h_hbm, maxa_hbm, tgt_hbm,
               num0, num1, den0, den1,
               acc_num, acc_den,
               smap, tgt_all, csrc, cdst, cslot, in_s, in_d,
               idx_so_a, idx_do_a, slot_a, arow_s_a, arow_d_a, hrow_a, exb_a,
               m_buf, znum, zden, rep, gnum, gden, sem_a, sem_e):
    cid = lax.axis_index("c")
    sid = lax.axis_index("s")
    tile_base = (cid * NS + sid) * EDGES_PER_TILE

    sets = ((idx_so_a, idx_do_a, slot_a, arow_s_a, arow_d_a, hrow_a, exb_a,
             sem_a),)

    pltpu.sync_copy(maxa_hbm, m_buf)
    mvec = m_buf[0, pl.ds(0, 16)]

    pltpu.sync_copy(tgt_hbm, tgt_all)

    garb16 = jnp.full((16,), GARB, jnp.int32)
    z16i = jnp.zeros((16,), jnp.int32)
    z16 = jnp.zeros((16,), jnp.float32)
    lane16 = lax.iota(jnp.int32, 16)

    def zrow(i, _):
        for j in range(D // 16):
            znum[i, pl.ds(16 * j, 16)] = z16
        zden[i, pl.ds(0, 16)] = z16
        return 0

    lax.fori_loop(0, SROWS // 3, zrow, 0)

    def fillg(i, _):
        smap[pl.ds(16 * i, 16)] = garb16
        return 0

    lax.fori_loop(0, N // 16, fillg, 0)

    def scat(i, _):
        tvec = tgt_all[pl.ds(16 * i, 16)]
        plsc.store_scatter(smap, [tvec], lane16 + 16 * i)
        return 0

    lax.fori_loop(0, TGT // 16, scat, 0)

    for i in range(TPT // 16):
        tvec = tgt_all[pl.ds(sid * TPT + 16 * i, 16)]
        rep[pl.ds(16 * i, 16)] = plsc.load_gather(smap, [tvec])

    def comp_chunk(k, wpos):
        pltpu.sync_copy(src_hbm.at[pl.ds(tile_base + k * CC, CC)], in_s)
        pltpu.sync_copy(dst_hbm.at[pl.ds(tile_base + k * CC, CC)], in_d)

        def inner(i, w):
            svec = in_s[pl.ds(16 * i, 16)]
            dvec = in_d[pl.ds(16 * i, 16)]
            sl = plsc.load_gather(smap, [dvec])
            mask = sl < GARB
            plsc.store_compressed(csrc.at[pl.ds(w, 16)], svec, mask=mask)
            plsc.store_compressed(cdst.at[pl.ds(w, 16)], dvec, mask=mask)
            plsc.store_compressed(cslot.at[pl.ds(w, 16)], sl, mask=mask)
            cnt = plsc.all_reduce_population_count(mask)
            return w + cnt[0]

        return lax.fori_loop(0, CC // 16, inner, wpos)

    wpos = lax.fori_loop(0, EDGES_PER_TILE // CC, comp_chunk, 0)

    nch = 2 * ((wpos + 2 * C - 1) // (2 * C))
    npad16 = (nch * C - wpos + 15) // 16

    def padf(j, _):
        base = wpos + 16 * j
        csrc[pl.ds(base, 16)] = z16i
        cdst[pl.ds(base, 16)] = z16i
        cslot[pl.ds(base, 16)] = garb16
        return 0

    lax.fori_loop(0, npad16, padf, 0)

    def t_body(t, _):
        for j in range(3):
            off = sid * SROWS + j * (SROWS // 3)
            pltpu.sync_copy(znum, acc_num.at[pl.ds(off, SROWS // 3), :])
            pltpu.sync_copy(zden, acc_den.at[pl.ds(off, SROWS // 3), :])
        plsc.subcore_barrier()

        tN = t * N

        def fire(k, s):
            idx_so, idx_do, slot_c, arow_s, arow_d, hrow, exb, sem = sets[s]
            base = k * C

            def offs(i, _):
                idx_so[pl.ds(16 * i, 16)] = csrc[pl.ds(base + 16 * i, 16)] + tN
                idx_do[pl.ds(16 * i, 16)] = cdst[pl.ds(base + 16 * i, 16)] + tN
                slot_c[pl.ds(16 * i, 16)] = cslot[pl.ds(base + 16 * i, 16)]
                return 0

            lax.fori_loop(0, C // 16, offs, 0)
            pltpu.async_copy(as_hbm.at[idx_so], arow_s, sem)
            pltpu.async_copy(ad_hbm.at[idx_do], arow_d, sem)
            pltpu.async_copy(h_hbm.at[idx_so], hrow, sem)

        def finish(s):
            idx_so, idx_do, slot_c, arow_s, arow_d, hrow, exb, sem = sets[s]
            pltpu.make_async_copy(as_hbm.at[idx_so], arow_s, sem).wait()
            pltpu.make_async_copy(ad_hbm.at[idx_do], arow_d, sem).wait()
            pltpu.make_async_copy(h_hbm.at[idx_so], hrow, sem).wait()

            def edge_body(i, _):
                sv = arow_s[i, :]
                dv = arow_d[i, :]
                z = sv + dv
                lr = jnp.maximum(z, 0.2 * z)
                zm = mvec + dv
                lrm = jnp.maximum(zm, 0.2 * zm)
                ex = jnp.exp(lr - lrm)
                exb[i, :] = ex
                for hh in range(H):
                    sc = ex[hh]
                    hv = hrow[i, pl.ds(16 * hh, 16)]
                    hrow[i, pl.ds(16 * hh, 16)] = sc * hv
                return 0

            lax.fori_loop(0, C, edge_body, 0)

            pltpu.sync_copy(hrow, acc_num.at[slot_c], add=True)
            pltpu.sync_copy(exb, acc_den.at[slot_c], add=True)

        def body1(k, _):
            fire(k, 0)
            finish(0)
            return 0

        lax.fori_loop(0, nch, body1, 0)
        plsc.subcore_barrier()

        cpn = pltpu.async_copy(acc_num.at[rep], gnum, sem_e)
        cpd = pltpu.async_copy(acc_den.at[rep], gden, sem_e)
        cpn.wait()
        cpd.wait()
        obase = t * TGT + sid * TPT

        @pl.when(cid == 0)
        def _():
            pltpu.sync_copy(gnum, num0.at[pl.ds(obase, TPT), :])
            pltpu.sync_copy(gden, den0.at[pl.ds(obase, TPT), :])

        @pl.when(cid == 1)
        def _():
            pltpu.sync_copy(gnum, num1.at[pl.ds(obase, TPT), :])
            pltpu.sync_copy(gden, den1.at[pl.ds(obase, TPT), :])

        plsc.subcore_barrier()
        return 0

    lax.fori_loop(0, T, t_body, 0)


@functools.cache
def _make_edge_call():
  return pl.kernel(
    _edge_body,
    out_type=[
        jax.ShapeDtypeStruct((T * TGT, D), jnp.float32),
        jax.ShapeDtypeStruct((T * TGT, D), jnp.float32),
        jax.ShapeDtypeStruct((T * TGT, 16), jnp.float32),
        jax.ShapeDtypeStruct((T * TGT, 16), jnp.float32),
    ],
    mesh=plsc.VectorSubcoreMesh(core_axis_name="c", subcore_axis_name="s",
                                num_cores=NC, num_subcores=NS),
    scratch_types=[
        pltpu.VMEM_SHARED((SLOTS, D), jnp.float32),
        pltpu.VMEM_SHARED((SLOTS, 16), jnp.float32),
        pltpu.VMEM((N,), jnp.int32),
        pltpu.VMEM((TGT,), jnp.int32),
        pltpu.VMEM((CBUF,), jnp.int32),
        pltpu.VMEM((CBUF,), jnp.int32),
        pltpu.VMEM((CBUF,), jnp.int32),
        pltpu.VMEM((CC,), jnp.int32),
        pltpu.VMEM((CC,), jnp.int32),
        pltpu.VMEM((C,), jnp.int32),
        pltpu.VMEM((C,), jnp.int32),
        pltpu.VMEM((C,), jnp.int32),
        pltpu.VMEM((C, 16), jnp.float32),
        pltpu.VMEM((C, 16), jnp.float32),
        pltpu.VMEM((C, D), jnp.float32),
        pltpu.VMEM((C, 16), jnp.float32),
        pltpu.VMEM((1, 128), jnp.float32),
        pltpu.VMEM((SROWS // 3, D), jnp.float32),
        pltpu.VMEM((SROWS // 3, 16), jnp.float32),
        pltpu.VMEM((TPT,), jnp.int32),
        pltpu.VMEM((TPT, D), jnp.float32),
        pltpu.VMEM((TPT, 16), jnp.float32),
        pltpu.SemaphoreType.DMA,
        pltpu.SemaphoreType.DMA,
    ],
    compiler_params=pltpu.CompilerParams(use_tc_tiling_on_sc=False,
                                         needs_layout_passes=False),
  )


def _temporal_body(n0, n1, d0, d1, pos, wq, wk, wv, rh, rs, out):
    num2 = (n0[...] + n1[...]).reshape(T * BLK3, D)
    den2 = (d0[...] + d1[...]).reshape(T * BLK3, 16)
    den128 = jnp.dot(den2, rh[...], preferred_element_type=jnp.float32) + 1e-16
    g = num2 / den128
    g = jnp.where(g > 0, g, jnp.exp(g) - 1.0)
    s3 = g.reshape(T, BLK3, D) + pos[...][:, None, :]
    s2 = s3.reshape(T * BLK3, D)
    k2 = jnp.dot(s2, wk[...], preferred_element_type=jnp.float32).reshape(T, BLK3, D)
    v2 = jnp.dot(s2, wv[...], preferred_element_type=jnp.float32).reshape(T, BLK3, D)
    q7 = jnp.dot(s3[T - 1], wq[...], preferred_element_type=jnp.float32)
    scs = [jnp.dot(q7 * k2[t], rs[...], preferred_element_type=jnp.float32) * 0.25
           for t in range(T)]
    m = scs[0]
    for t in range(1, T):
        m = jnp.maximum(m, scs[t])
    exs = [jnp.exp(s - m) for s in scs]
    dsum = exs[0]
    for t in range(1, T):
        dsum = dsum + exs[t]
    rh8 = rh[...][0:8, :]
    tout = jnp.zeros((BLK3, D), jnp.float32)
    for t in range(T):
        w_t = exs[t] / dsum
        tout = tout + jnp.dot(w_t, rh8, preferred_element_type=jnp.float32) * v2[t]
    out[...] = tout + s3[T - 1]


def _temporal_call(num0, num1, den0, den1, pos_emb, Wq, Wk, Wv, RH, RS):
    nblk = TGT // BLK3
    return pl.pallas_call(
        _temporal_body,
        grid=(nblk,),
        in_specs=[
            pl.BlockSpec((T, BLK3, D), lambda b: (0, b, 0)),
            pl.BlockSpec((T, BLK3, D), lambda b: (0, b, 0)),
            pl.BlockSpec((T, BLK3, 16), lambda b: (0, b, 0)),
            pl.BlockSpec((T, BLK3, 16), lambda b: (0, b, 0)),
            pl.BlockSpec((T, D), lambda b: (0, 0)),
            pl.BlockSpec((D, D), lambda b: (0, 0)),
            pl.BlockSpec((D, D), lambda b: (0, 0)),
            pl.BlockSpec((D, D), lambda b: (0, 0)),
            pl.BlockSpec((16, D), lambda b: (0, 0)),
            pl.BlockSpec((D, 8), lambda b: (0, 0)),
        ],
        out_specs=pl.BlockSpec((BLK3, D), lambda b: (b, 0)),
        out_shape=jax.ShapeDtypeStruct((TGT, D), jnp.float32),
    )(num0, num1, den0, den1, pos_emb, Wq, Wk, Wv, RH, RS)


def _cls_body(e_ref, w1, b1, w2, b2, out):
    e = e_ref[...]
    p = e[0:1024] * e[1024:2048]
    h1 = jnp.dot(p, w1[...], preferred_element_type=jnp.float32) + b1[...]
    h1 = jnp.maximum(h1, 0.0)
    out[...] = jnp.dot(h1, w2[...], preferred_element_type=jnp.float32) + b2[...]


def _cls_call(emb, Wc1, bc1, Wc2, bc2):
    return pl.pallas_call(
        _cls_body,
        out_shape=jax.ShapeDtypeStruct((TGT // 2, 2), jnp.float32),
    )(emb, Wc1, bc1.reshape(1, D), Wc2, bc2.reshape(1, 2))


def kernel(graphs_x, edge_index, nids, W_s, a_src, a_dst, Wq, Wk, Wv,
           pos_emb, Wc1, bc1, Wc2, bc2):
    x2 = graphs_x.reshape(T * N, F)
    src = edge_index[0].astype(jnp.int32)
    dst = edge_index[1].astype(jnp.int32)
    tgt = jnp.concatenate([nids[:, 0], nids[:, 1]]).astype(jnp.int32)

    head_of_lane = jnp.arange(D) // DH
    blockdiag = (head_of_lane[:, None] == jnp.arange(H)[None, :]).astype(jnp.float32)
    Asrc = blockdiag * a_src.reshape(D)[:, None]
    Adst = blockdiag * a_dst.reshape(D)[:, None]
    A128 = jnp.concatenate(
        [Asrc, Asrc, Adst, Adst, jnp.zeros((D, 96), jnp.float32)], axis=1)
    RH = (jnp.arange(16)[:, None] == head_of_lane[None, :]).astype(jnp.float32)
    RS = RH[:8].T

    h_all, alpha_s, alpha_d, maxa = _prep_call(x2, W_s, A128)
    num0, num1, den0, den1 = _make_edge_call()(src, dst, alpha_s, alpha_d,
                                               h_all, maxa, tgt)
    emb = _temporal_call(num0.reshape(T, TGT, D), num1.reshape(T, TGT, D),
                         den0.reshape(T, TGT, 16), den1.reshape(T, TGT, 16),
                         pos_emb, Wq, Wk, Wv, RH, RS)
    return _cls_call(emb, Wc1, bc1, Wc2, bc2)


# speedup vs baseline: 1.7249x; 1.7249x over previous
import functools

import jax
import jax.numpy as jnp
from jax import lax
from jax.experimental import pallas as pl
from jax.experimental.pallas import tpu as pltpu
from jax.experimental.pallas import tpu_sc as plsc

N = 10000
T = 8
F = 128
E = 320000
D = 128
H = 8
DH = 16

NC = 2
NS = 16
EDGES_PER_TILE = E // (NC * NS)
C = 128
CC = 400
TGT = 2048
TPT = TGT // NS
GARB = TGT
SLOTS = TGT + 16
SROWS = SLOTS // NS
CBUF = EDGES_PER_TILE + 2 * C + 16

BLK1 = 2000
BLK3 = 256


def _prep_body(x_ref, w_ref, a_ref, h_ref, as_ref, ad_ref, maxa_ref):
    b = pl.program_id(0)
    hb = jnp.dot(x_ref[...], w_ref[...], preferred_element_type=jnp.float32)
    ab = jnp.dot(hb, a_ref[...], preferred_element_type=jnp.float32)
    h_ref[...] = hb
    as_ref[...] = ab[:, 0:16]
    ad_ref[...] = ab[:, 16:32]
    lane = lax.broadcasted_iota(jnp.int32, (1, 128), 1)
    colmax = jnp.max(ab, axis=0, keepdims=True)
    m = jnp.where(lane < 16, colmax, 0.0)

    @pl.when(b == 0)
    def _():
        maxa_ref[...] = m

    @pl.when(b != 0)
    def _():
        maxa_ref[...] = jnp.maximum(maxa_ref[...], m)


def _prep_call(x2, W_s, A128):
    nblk = (T * N) // BLK1
    return pl.pallas_call(
        _prep_body,
        grid=(nblk,),
        in_specs=[
            pl.BlockSpec((BLK1, F), lambda b: (b, 0)),
            pl.BlockSpec((F, D), lambda b: (0, 0)),
            pl.BlockSpec((D, 128), lambda b: (0, 0)),
        ],
        out_specs=[
            pl.BlockSpec((BLK1, D), lambda b: (b, 0)),
            pl.BlockSpec((BLK1, 16), lambda b: (b, 0)),
            pl.BlockSpec((BLK1, 16), lambda b: (b, 0)),
            pl.BlockSpec((1, 128), lambda b: (0, 0)),
        ],
        out_shape=[
            jax.ShapeDtypeStruct((T * N, D), jnp.float32),
            jax.ShapeDtypeStruct((T * N, 16), jnp.float32),
            jax.ShapeDtypeStruct((T * N, 16), jnp.float32),
            jax.ShapeDtypeStruct((1, 128), jnp.float32),
        ],
    )(x2, W_s, A128)


def _edge_body(src_hbm, dst_hbm, as_hbm, ad_hbm, h_hbm, maxa_hbm, tgt_hbm,
               num0, num1, den0, den1,
               acc_num, acc_den,
               smap, tgt_all, csrc, cdst, cslot, in_s, in_d,
               idx_so_a, idx_do_a, slot_a, arow_s_a, arow_d_a, hrow_a, exb_a,
               m_buf, znum, zden, rep, gnum, gden, sem_a, sem_e):
    cid = lax.axis_index("c")
    sid = lax.axis_index("s")
    tile_base = (cid * NS + sid) * EDGES_PER_TILE

    sets = ((idx_so_a, idx_do_a, slot_a, arow_s_a, arow_d_a, hrow_a, exb_a,
             sem_a),)

    pltpu.sync_copy(maxa_hbm, m_buf)
    mvec = m_buf[0, pl.ds(0, 16)]

    pltpu.sync_copy(tgt_hbm, tgt_all)

    garb16 = jnp.full((16,), GARB, jnp.int32)
    z16i = jnp.zeros((16,), jnp.int32)
    z16 = jnp.zeros((16,), jnp.float32)
    lane16 = lax.iota(jnp.int32, 16)

    def zrow(i, _):
        for j in range(D // 16):
            znum[i, pl.ds(16 * j, 16)] = z16
        zden[i, pl.ds(0, 16)] = z16
        return 0

    lax.fori_loop(0, SROWS // 3, zrow, 0)

    def fillg(i, _):
        smap[pl.ds(16 * i, 16)] = garb16
        return 0

    lax.fori_loop(0, N // 16, fillg, 0)

    def scat(i, _):
        tvec = tgt_all[pl.ds(16 * i, 16)]
        plsc.store_scatter(smap, [tvec], lane16 + 16 * i)
        return 0

    lax.fori_loop(0, TGT // 16, scat, 0)

    for i in range(TPT // 16):
        tvec = tgt_all[pl.ds(sid * TPT + 16 * i, 16)]
        rep[pl.ds(16 * i, 16)] = plsc.load_gather(smap, [tvec])

    def comp_chunk(k, wpos):
        pltpu.sync_copy(src_hbm.at[pl.ds(tile_base + k * CC, CC)], in_s)
        pltpu.sync_copy(dst_hbm.at[pl.ds(tile_base + k * CC, CC)], in_d)

        def inner(i, w):
            svec = in_s[pl.ds(16 * i, 16)]
            dvec = in_d[pl.ds(16 * i, 16)]
            sl = plsc.load_gather(smap, [dvec])
            mask = sl < GARB
            plsc.store_compressed(csrc.at[pl.ds(w, 16)], svec, mask=mask)
            plsc.store_compressed(cdst.at[pl.ds(w, 16)], dvec, mask=mask)
            plsc.store_compressed(cslot.at[pl.ds(w, 16)], sl, mask=mask)
            cnt = plsc.all_reduce_population_count(mask)
            return w + cnt[0]

        return lax.fori_loop(0, CC // 16, inner, wpos)

    wpos = lax.fori_loop(0, EDGES_PER_TILE // CC, comp_chunk, 0)

    nch = 2 * ((wpos + 2 * C - 1) // (2 * C))
    npad16 = (nch * C - wpos + 15) // 16

    def padf(j, _):
        base = wpos + 16 * j
        csrc[pl.ds(base, 16)] = z16i
        cdst[pl.ds(base, 16)] = z16i
        cslot[pl.ds(base, 16)] = garb16
        return 0

    lax.fori_loop(0, npad16, padf, 0)

    def t_body(t, _):
        for j in range(3):
            off = sid * SROWS + j * (SROWS // 3)
            pltpu.sync_copy(znum, acc_num.at[pl.ds(off, SROWS // 3), :])
            pltpu.sync_copy(zden, acc_den.at[pl.ds(off, SROWS // 3), :])
        plsc.subcore_barrier()

        tN = t * N

        def fire(k, s):
            idx_so, idx_do, slot_c, arow_s, arow_d, hrow, exb, sem = sets[s]
            base = k * C

            def offs(i, _):
                idx_so[pl.ds(16 * i, 16)] = csrc[pl.ds(base + 16 * i, 16)] + tN
                idx_do[pl.ds(16 * i, 16)] = cdst[pl.ds(base + 16 * i, 16)] + tN
                slot_c[pl.ds(16 * i, 16)] = cslot[pl.ds(base + 16 * i, 16)]
                return 0

            lax.fori_loop(0, C // 16, offs, 0)
            pltpu.async_copy(as_hbm.at[idx_so], arow_s, sem)
            pltpu.async_copy(ad_hbm.at[idx_do], arow_d, sem)
            pltpu.async_copy(h_hbm.at[idx_so], hrow, sem)

        def finish(s):
            idx_so, idx_do, slot_c, arow_s, arow_d, hrow, exb, sem = sets[s]
            pltpu.make_async_copy(as_hbm.at[idx_so], arow_s, sem).wait()
            pltpu.make_async_copy(ad_hbm.at[idx_do], arow_d, sem).wait()
            pltpu.make_async_copy(h_hbm.at[idx_so], hrow, sem).wait()

            def edge_body(i, _):
                sv = arow_s[i, :]
                dv = arow_d[i, :]
                z = sv + dv
                lr = jnp.maximum(z, 0.2 * z)
                zm = mvec + dv
                lrm = jnp.maximum(zm, 0.2 * zm)
                ex = jnp.exp(lr - lrm)
                exb[i, :] = ex
                for hh in range(H):
                    sc = ex[hh]
                    hv = hrow[i, pl.ds(16 * hh, 16)]
                    hrow[i, pl.ds(16 * hh, 16)] = sc * hv
                return 0

            lax.fori_loop(0, C, edge_body, 0)

            pltpu.sync_copy(hrow, acc_num.at[slot_c], add=True)
            pltpu.sync_copy(exb, acc_den.at[slot_c], add=True)

        def body1(k, _):
            fire(k, 0)
            finish(0)
            return 0

        lax.fori_loop(0, nch, body1, 0)
        plsc.subcore_barrier()

        cpn = pltpu.async_copy(acc_num.at[rep], gnum, sem_e)
        cpd = pltpu.async_copy(acc_den.at[rep], gden, sem_e)
        cpn.wait()
        cpd.wait()
        obase = t * TGT + sid * TPT

        @pl.when(cid == 0)
        def _():
            pltpu.sync_copy(gnum, num0.at[pl.ds(obase, TPT), :])
            pltpu.sync_copy(gden, den0.at[pl.ds(obase, TPT), :])

        @pl.when(cid == 1)
        def _():
            pltpu.sync_copy(gnum, num1.at[pl.ds(obase, TPT), :])
            pltpu.sync_copy(gden, den1.at[pl.ds(obase, TPT), :])

        plsc.subcore_barrier()
        return 0

    lax.fori_loop(0, T, t_body, 0)


@functools.cache
def _make_edge_call():
  return pl.kernel(
    _edge_body,
    out_type=[
        jax.ShapeDtypeStruct((T * TGT, D), jnp.float32),
        jax.ShapeDtypeStruct((T * TGT, D), jnp.float32),
        jax.ShapeDtypeStruct((T * TGT, 16), jnp.float32),
        jax.ShapeDtypeStruct((T * TGT, 16), jnp.float32),
    ],
    mesh=plsc.VectorSubcoreMesh(core_axis_name="c", subcore_axis_name="s",
                                num_cores=NC, num_subcores=NS),
    scratch_types=[
        pltpu.VMEM_SHARED((SLOTS, D), jnp.float32),
        pltpu.VMEM_SHARED((SLOTS, 16), jnp.float32),
        pltpu.VMEM((N,), jnp.int32),
        pltpu.VMEM((TGT,), jnp.int32),
        pltpu.VMEM((CBUF,), jnp.int32),
        pltpu.VMEM((CBUF,), jnp.int32),
        pltpu.VMEM((CBUF,), jnp.int32),
        pltpu.VMEM((CC,), jnp.int32),
        pltpu.VMEM((CC,), jnp.int32),
        pltpu.VMEM((C,), jnp.int32),
        pltpu.VMEM((C,), jnp.int32),
        pltpu.VMEM((C,), jnp.int32),
        pltpu.VMEM((C, 16), jnp.float32),
        pltpu.VMEM((C, 16), jnp.float32),
        pltpu.VMEM((C, D), jnp.float32),
        pltpu.VMEM((C, 16), jnp.float32),
        pltpu.VMEM((1, 128), jnp.float32),
        pltpu.VMEM((SROWS // 3, D), jnp.float32),
        pltpu.VMEM((SROWS // 3, 16), jnp.float32),
        pltpu.VMEM((TPT,), jnp.int32),
        pltpu.VMEM((TPT, D), jnp.float32),
        pltpu.VMEM((TPT, 16), jnp.float32),
        pltpu.SemaphoreType.DMA,
        pltpu.SemaphoreType.DMA,
    ],
    compiler_params=pltpu.CompilerParams(use_tc_tiling_on_sc=False,
                                         needs_layout_passes=False),
  )


def _temporal_body(n0, n1, d0, d1, pos, wq, wk, wv, rh, rs, out):
    num2 = (n0[...] + n1[...]).reshape(T * BLK3, D)
    den2 = (d0[...] + d1[...]).reshape(T * BLK3, 16)
    den128 = jnp.dot(den2, rh[...], preferred_element_type=jnp.float32) + 1e-16
    g = num2 / den128
    g = jnp.where(g > 0, g, jnp.exp(g) - 1.0)
    s3 = g.reshape(T, BLK3, D) + pos[...][:, None, :]
    s2 = s3.reshape(T * BLK3, D)
    k2 = jnp.dot(s2, wk[...], preferred_element_type=jnp.float32).reshape(T, BLK3, D)
    v2 = jnp.dot(s2, wv[...], preferred_element_type=jnp.float32).reshape(T, BLK3, D)
    q7 = jnp.dot(s3[T - 1], wq[...], preferred_element_type=jnp.float32)
    scs = [jnp.dot(q7 * k2[t], rs[...], preferred_element_type=jnp.float32) * 0.25
           for t in range(T)]
    m = scs[0]
    for t in range(1, T):
        m = jnp.maximum(m, scs[t])
    exs = [jnp.exp(s - m) for s in scs]
    dsum = exs[0]
    for t in range(1, T):
        dsum = dsum + exs[t]
    rh8 = rh[...][0:8, :]
    tout = jnp.zeros((BLK3, D), jnp.float32)
    for t in range(T):
        w_t = exs[t] / dsum
        tout = tout + jnp.dot(w_t, rh8, preferred_element_type=jnp.float32) * v2[t]
    out[...] = tout + s3[T - 1]


def _temporal_call(num0, num1, den0, den1, pos_emb, Wq, Wk, Wv, RH, RS):
    nblk = TGT // BLK3
    return pl.pallas_call(
        _temporal_body,
        grid=(nblk,),
        in_specs=[
            pl.BlockSpec((T, BLK3, D), lambda b: (0, b, 0)),
            pl.BlockSpec((T, BLK3, D), lambda b: (0, b, 0)),
            pl.BlockSpec((T, BLK3, 16), lambda b: (0, b, 0)),
            pl.BlockSpec((T, BLK3, 16), lambda b: (0, b, 0)),
            pl.BlockSpec((T, D), lambda b: (0, 0)),
            pl.BlockSpec((D, D), lambda b: (0, 0)),
            pl.BlockSpec((D, D), lambda b: (0, 0)),
            pl.BlockSpec((D, D), lambda b: (0, 0)),
            pl.BlockSpec((16, D), lambda b: (0, 0)),
            pl.BlockSpec((D, 8), lambda b: (0, 0)),
        ],
        out_specs=pl.BlockSpec((BLK3, D), lambda b: (b, 0)),
        out_shape=jax.ShapeDtypeStruct((TGT, D), jnp.float32),
    )(num0, num1, den0, den1, pos_emb, Wq, Wk, Wv, RH, RS)


def _cls_body(e_ref, w1, b1, w2, b2, out):
    e = e_ref[...]
    p = e[0:1024] * e[1024:2048]
    h1 = jnp.dot(p, w1[...], preferred_element_type=jnp.float32) + b1[...]
    h1 = jnp.maximum(h1, 0.0)
    out[...] = jnp.dot(h1, w2[...], preferred_element_type=jnp.float32) + b2[...]


def _cls_call(emb, Wc1, bc1, Wc2, bc2):
    return pl.pallas_call(
        _cls_body,
        out_shape=jax.ShapeDtypeStruct((TGT // 2, 2), jnp.float32),
    )(emb, Wc1, bc1.reshape(1, D), Wc2, bc2.reshape(1, 2))


def kernel(graphs_x, edge_index, nids, W_s, a_src, a_dst, Wq, Wk, Wv,
           pos_emb, Wc1, bc1, Wc2, bc2):
    x2 = graphs_x.reshape(T * N, F)
    src = edge_index[0].astype(jnp.int32)
    dst = edge_index[1].astype(jnp.int32)
    tgt = jnp.concatenate([nids[:, 0], nids[:, 1]]).astype(jnp.int32)

    head_of_lane = jnp.arange(D) // DH
    blockdiag = (head_of_lane[:, None] == jnp.arange(H)[None, :]).astype(jnp.float32)
    Asrc = blockdiag * a_src.reshape(D)[:, None]
    Adst = blockdiag * a_dst.reshape(D)[:, None]
    A128 = jnp.concatenate(
        [Asrc, Asrc, Adst, Adst, jnp.zeros((D, 96), jnp.float32)], axis=1)
    RH = (jnp.arange(16)[:, None] == head_of_lane[None, :]).astype(jnp.float32)
    RS = RH[:8].T

    h_all, alpha_s, alpha_d, maxa = _prep_call(x2, W_s, A128)
    num0, num1, den0, den1 = _make_edge_call()(src, dst, alpha_s, alpha_d,
                                               h_all, maxa, tgt)
    emb = _temporal_call(num0.reshape(T, TGT, D), num1.reshape(T, TGT, D),
                         den0.reshape(T, TGT, 16), den1.reshape(T, TGT, 16),
                         pos_emb, Wq, Wk, Wv, RH, RS)
    return _cls_call(emb, Wc1, bc1, Wc2, bc2)


# speedup vs baseline: 2.4038x; 1.3936x over previous
import functools

import jax
import jax.numpy as jnp
from jax import lax
from jax.experimental import pallas as pl
from jax.experimental.pallas import tpu as pltpu
from jax.experimental.pallas import tpu_sc as plsc

N = 10000
T = 8
F = 128
E = 320000
D = 128
H = 8
DH = 16

NC = 2
NS = 16
EDGES_PER_TILE = E // (NC * NS)
C = 160
CC = 400
TGT = 2048
TPT = TGT // NS
GARB = TGT
SLOTS = TGT + 16
SROWS = SLOTS // NS
CBUF = EDGES_PER_TILE + 2 * C + 16

BLK1 = 2000
BLK3 = 256


def _prep_body(x_ref, w_ref, a_ref, h_ref, as_ref, ad_ref, maxa_ref):
    b = pl.program_id(0)
    hb = jnp.dot(x_ref[...], w_ref[...], preferred_element_type=jnp.float32)
    ab = jnp.dot(hb, a_ref[...], preferred_element_type=jnp.float32)
    h_ref[...] = hb
    as_ref[...] = ab[:, 0:16]
    ad_ref[...] = ab[:, 16:32]
    lane = lax.broadcasted_iota(jnp.int32, (1, 128), 1)
    colmax = jnp.max(ab, axis=0, keepdims=True)
    m = jnp.where(lane < 16, colmax, 0.0)

    @pl.when(b == 0)
    def _():
        maxa_ref[...] = m

    @pl.when(b != 0)
    def _():
        maxa_ref[...] = jnp.maximum(maxa_ref[...], m)


def _prep_call(x2, W_s, A128):
    nblk = (T * N) // BLK1
    return pl.pallas_call(
        _prep_body,
        grid=(nblk,),
        in_specs=[
            pl.BlockSpec((BLK1, F), lambda b: (b, 0)),
            pl.BlockSpec((F, D), lambda b: (0, 0)),
            pl.BlockSpec((D, 128), lambda b: (0, 0)),
        ],
        out_specs=[
            pl.BlockSpec((BLK1, D), lambda b: (b, 0)),
            pl.BlockSpec((BLK1, 16), lambda b: (b, 0)),
            pl.BlockSpec((BLK1, 16), lambda b: (b, 0)),
            pl.BlockSpec((1, 128), lambda b: (0, 0)),
        ],
        out_shape=[
            jax.ShapeDtypeStruct((T * N, D), jnp.float32),
            jax.ShapeDtypeStruct((T * N, 16), jnp.float32),
            jax.ShapeDtypeStruct((T * N, 16), jnp.float32),
            jax.ShapeDtypeStruct((1, 128), jnp.float32),
        ],
    )(x2, W_s, A128)


def _edge_body(src_hbm, dst_hbm, as_hbm, ad_hbm, h_hbm, maxa_hbm, tgt_hbm,
               num0, num1, den0, den1,
               acc_num, acc_den,
               smap, tgt_all, csrc, cdst, cslot, in_s, in_d,
               idx_so_a, idx_do_a, slot_a, arow_s_a, arow_d_a, hrow_a, exb_a,
               m_buf, znum, zden, rep, gnum, gden, sem_a, sem_e):
    cid = lax.axis_index("c")
    sid = lax.axis_index("s")
    tile_base = (cid * NS + sid) * EDGES_PER_TILE

    sets = ((idx_so_a, idx_do_a, slot_a, arow_s_a, arow_d_a, hrow_a, exb_a,
             sem_a),)

    pltpu.sync_copy(maxa_hbm, m_buf)
    mvec = m_buf[0, pl.ds(0, 16)]

    pltpu.sync_copy(tgt_hbm, tgt_all)

    garb16 = jnp.full((16,), GARB, jnp.int32)
    z16i = jnp.zeros((16,), jnp.int32)
    z16 = jnp.zeros((16,), jnp.float32)
    lane16 = lax.iota(jnp.int32, 16)

    def zrow(i, _):
        for j in range(D // 16):
            znum[i, pl.ds(16 * j, 16)] = z16
        zden[i, pl.ds(0, 16)] = z16
        return 0

    lax.fori_loop(0, SROWS // 3, zrow, 0)

    def fillg(i, _):
        smap[pl.ds(16 * i, 16)] = garb16
        return 0

    lax.fori_loop(0, N // 16, fillg, 0)

    def scat(i, _):
        tvec = tgt_all[pl.ds(16 * i, 16)]
        plsc.store_scatter(smap, [tvec], lane16 + 16 * i)
        return 0

    lax.fori_loop(0, TGT // 16, scat, 0)

    for i in range(TPT // 16):
        tvec = tgt_all[pl.ds(sid * TPT + 16 * i, 16)]
        rep[pl.ds(16 * i, 16)] = plsc.load_gather(smap, [tvec])

    def comp_chunk(k, wpos):
        pltpu.sync_copy(src_hbm.at[pl.ds(tile_base + k * CC, CC)], in_s)
        pltpu.sync_copy(dst_hbm.at[pl.ds(tile_base + k * CC, CC)], in_d)

        def inner(i, w):
            svec = in_s[pl.ds(16 * i, 16)]
            dvec = in_d[pl.ds(16 * i, 16)]
            sl = plsc.load_gather(smap, [dvec])
            mask = sl < GARB
            plsc.store_compressed(csrc.at[pl.ds(w, 16)], svec, mask=mask)
            plsc.store_compressed(cdst.at[pl.ds(w, 16)], dvec, mask=mask)
            plsc.store_compressed(cslot.at[pl.ds(w, 16)], sl, mask=mask)
            cnt = plsc.all_reduce_population_count(mask)
            return w + cnt[0]

        return lax.fori_loop(0, CC // 16, inner, wpos)

    wpos = lax.fori_loop(0, EDGES_PER_TILE // CC, comp_chunk, 0)

    nch = 2 * ((wpos + 2 * C - 1) // (2 * C))
    npad16 = (nch * C - wpos + 15) // 16

    def padf(j, _):
        base = wpos + 16 * j
        csrc[pl.ds(base, 16)] = z16i
        cdst[pl.ds(base, 16)] = z16i
        cslot[pl.ds(base, 16)] = garb16
        return 0

    lax.fori_loop(0, npad16, padf, 0)

    def t_body(t, _):
        for j in range(3):
            off = sid * SROWS + j * (SROWS // 3)
            pltpu.sync_copy(znum, acc_num.at[pl.ds(off, SROWS // 3), :])
            pltpu.sync_copy(zden, acc_den.at[pl.ds(off, SROWS // 3), :])
        plsc.subcore_barrier()

        tN = t * N

        def fire(k, s):
            idx_so, idx_do, slot_c, arow_s, arow_d, hrow, exb, sem = sets[s]
            base = k * C

            def offs(i, _):
                idx_so[pl.ds(16 * i, 16)] = csrc[pl.ds(base + 16 * i, 16)] + tN
                idx_do[pl.ds(16 * i, 16)] = cdst[pl.ds(base + 16 * i, 16)] + tN
                slot_c[pl.ds(16 * i, 16)] = cslot[pl.ds(base + 16 * i, 16)]
                return 0

            lax.fori_loop(0, C // 16, offs, 0)
            pltpu.async_copy(as_hbm.at[idx_so], arow_s, sem)
            pltpu.async_copy(ad_hbm.at[idx_do], arow_d, sem)
            pltpu.async_copy(h_hbm.at[idx_so], hrow, sem)

        def finish(s):
            idx_so, idx_do, slot_c, arow_s, arow_d, hrow, exb, sem = sets[s]
            pltpu.make_async_copy(as_hbm.at[idx_so], arow_s, sem).wait()
            pltpu.make_async_copy(ad_hbm.at[idx_do], arow_d, sem).wait()
            pltpu.make_async_copy(h_hbm.at[idx_so], hrow, sem).wait()

            def edge_body(i, _):
                sv = arow_s[i, :]
                dv = arow_d[i, :]
                z = sv + dv
                lr = jnp.maximum(z, 0.2 * z)
                zm = mvec + dv
                lrm = jnp.maximum(zm, 0.2 * zm)
                ex = jnp.exp(lr - lrm)
                exb[i, :] = ex
                for hh in range(H):
                    sc = ex[hh]
                    hv = hrow[i, pl.ds(16 * hh, 16)]
                    hrow[i, pl.ds(16 * hh, 16)] = sc * hv
                return 0

            lax.fori_loop(0, C, edge_body, 0)

            pltpu.sync_copy(hrow, acc_num.at[slot_c], add=True)
            pltpu.sync_copy(exb, acc_den.at[slot_c], add=True)

        def body1(k, _):
            fire(k, 0)
            finish(0)
            return 0

        lax.fori_loop(0, nch, body1, 0)
        plsc.subcore_barrier()

        cpn = pltpu.async_copy(acc_num.at[rep], gnum, sem_e)
        cpd = pltpu.async_copy(acc_den.at[rep], gden, sem_e)
        cpn.wait()
        cpd.wait()
        obase = t * TGT + sid * TPT

        @pl.when(cid == 0)
        def _():
            pltpu.sync_copy(gnum, num0.at[pl.ds(obase, TPT), :])
            pltpu.sync_copy(gden, den0.at[pl.ds(obase, TPT), :])

        @pl.when(cid == 1)
        def _():
            pltpu.sync_copy(gnum, num1.at[pl.ds(obase, TPT), :])
            pltpu.sync_copy(gden, den1.at[pl.ds(obase, TPT), :])

        plsc.subcore_barrier()
        return 0

    lax.fori_loop(0, T, t_body, 0)


@functools.cache
def _make_edge_call():
  return pl.kernel(
    _edge_body,
    out_type=[
        jax.ShapeDtypeStruct((T * TGT, D), jnp.float32),
        jax.ShapeDtypeStruct((T * TGT, D), jnp.float32),
        jax.ShapeDtypeStruct((T * TGT, 16), jnp.float32),
        jax.ShapeDtypeStruct((T * TGT, 16), jnp.float32),
    ],
    mesh=plsc.VectorSubcoreMesh(core_axis_name="c", subcore_axis_name="s",
                                num_cores=NC, num_subcores=NS),
    scratch_types=[
        pltpu.VMEM_SHARED((SLOTS, D), jnp.float32),
        pltpu.VMEM_SHARED((SLOTS, 16), jnp.float32),
        pltpu.VMEM((N,), jnp.int32),
        pltpu.VMEM((TGT,), jnp.int32),
        pltpu.VMEM((CBUF,), jnp.int32),
        pltpu.VMEM((CBUF,), jnp.int32),
        pltpu.VMEM((CBUF,), jnp.int32),
        pltpu.VMEM((CC,), jnp.int32),
        pltpu.VMEM((CC,), jnp.int32),
        pltpu.VMEM((C,), jnp.int32),
        pltpu.VMEM((C,), jnp.int32),
        pltpu.VMEM((C,), jnp.int32),
        pltpu.VMEM((C, 16), jnp.float32),
        pltpu.VMEM((C, 16), jnp.float32),
        pltpu.VMEM((C, D), jnp.float32),
        pltpu.VMEM((C, 16), jnp.float32),
        pltpu.VMEM((1, 128), jnp.float32),
        pltpu.VMEM((SROWS // 3, D), jnp.float32),
        pltpu.VMEM((SROWS // 3, 16), jnp.float32),
        pltpu.VMEM((TPT,), jnp.int32),
        pltpu.VMEM((TPT, D), jnp.float32),
        pltpu.VMEM((TPT, 16), jnp.float32),
        pltpu.SemaphoreType.DMA,
        pltpu.SemaphoreType.DMA,
    ],
    compiler_params=pltpu.CompilerParams(use_tc_tiling_on_sc=False,
                                         needs_layout_passes=False),
  )


def _temporal_body(n0, n1, d0, d1, pos, wq, wk, wv, rh, rs, out):
    num2 = (n0[...] + n1[...]).reshape(T * BLK3, D)
    den2 = (d0[...] + d1[...]).reshape(T * BLK3, 16)
    den128 = jnp.dot(den2, rh[...], preferred_element_type=jnp.float32) + 1e-16
    g = num2 / den128
    g = jnp.where(g > 0, g, jnp.exp(g) - 1.0)
    s3 = g.reshape(T, BLK3, D) + pos[...][:, None, :]
    s2 = s3.reshape(T * BLK3, D)
    k2 = jnp.dot(s2, wk[...], preferred_element_type=jnp.float32).reshape(T, BLK3, D)
    v2 = jnp.dot(s2, wv[...], preferred_element_type=jnp.float32).reshape(T, BLK3, D)
    q7 = jnp.dot(s3[T - 1], wq[...], preferred_element_type=jnp.float32)
    scs = [jnp.dot(q7 * k2[t], rs[...], preferred_element_type=jnp.float32) * 0.25
           for t in range(T)]
    m = scs[0]
    for t in range(1, T):
        m = jnp.maximum(m, scs[t])
    exs = [jnp.exp(s - m) for s in scs]
    dsum = exs[0]
    for t in range(1, T):
        dsum = dsum + exs[t]
    rh8 = rh[...][0:8, :]
    tout = jnp.zeros((BLK3, D), jnp.float32)
    for t in range(T):
        w_t = exs[t] / dsum
        tout = tout + jnp.dot(w_t, rh8, preferred_element_type=jnp.float32) * v2[t]
    out[...] = tout + s3[T - 1]


def _temporal_call(num0, num1, den0, den1, pos_emb, Wq, Wk, Wv, RH, RS):
    nblk = TGT // BLK3
    return pl.pallas_call(
        _temporal_body,
        grid=(nblk,),
        in_specs=[
            pl.BlockSpec((T, BLK3, D), lambda b: (0, b, 0)),
            pl.BlockSpec((T, BLK3, D), lambda b: (0, b, 0)),
            pl.BlockSpec((T, BLK3, 16), lambda b: (0, b, 0)),
            pl.BlockSpec((T, BLK3, 16), lambda b: (0, b, 0)),
            pl.BlockSpec((T, D), lambda b: (0, 0)),
            pl.BlockSpec((D, D), lambda b: (0, 0)),
            pl.BlockSpec((D, D), lambda b: (0, 0)),
            pl.BlockSpec((D, D), lambda b: (0, 0)),
            pl.BlockSpec((16, D), lambda b: (0, 0)),
            pl.BlockSpec((D, 8), lambda b: (0, 0)),
        ],
        out_specs=pl.BlockSpec((BLK3, D), lambda b: (b, 0)),
        out_shape=jax.ShapeDtypeStruct((TGT, D), jnp.float32),
    )(num0, num1, den0, den1, pos_emb, Wq, Wk, Wv, RH, RS)


def _cls_body(e_ref, w1, b1, w2, b2, out):
    e = e_ref[...]
    p = e[0:1024] * e[1024:2048]
    h1 = jnp.dot(p, w1[...], preferred_element_type=jnp.float32) + b1[...]
    h1 = jnp.maximum(h1, 0.0)
    out[...] = jnp.dot(h1, w2[...], preferred_element_type=jnp.float32) + b2[...]


def _cls_call(emb, Wc1, bc1, Wc2, bc2):
    return pl.pallas_call(
        _cls_body,
        out_shape=jax.ShapeDtypeStruct((TGT // 2, 2), jnp.float32),
    )(emb, Wc1, bc1.reshape(1, D), Wc2, bc2.reshape(1, 2))


def kernel(graphs_x, edge_index, nids, W_s, a_src, a_dst, Wq, Wk, Wv,
           pos_emb, Wc1, bc1, Wc2, bc2):
    x2 = graphs_x.reshape(T * N, F)
    src = edge_index[0].astype(jnp.int32)
    dst = edge_index[1].astype(jnp.int32)
    tgt = jnp.concatenate([nids[:, 0], nids[:, 1]]).astype(jnp.int32)

    head_of_lane = jnp.arange(D) // DH
    blockdiag = (head_of_lane[:, None] == jnp.arange(H)[None, :]).astype(jnp.float32)
    Asrc = blockdiag * a_src.reshape(D)[:, None]
    Adst = blockdiag * a_dst.reshape(D)[:, None]
    A128 = jnp.concatenate(
        [Asrc, Asrc, Adst, Adst, jnp.zeros((D, 96), jnp.float32)], axis=1)
    RH = (jnp.arange(16)[:, None] == head_of_lane[None, :]).astype(jnp.float32)
    RS = RH[:8].T

    h_all, alpha_s, alpha_d, maxa = _prep_call(x2, W_s, A128)
    num0, num1, den0, den1 = _make_edge_call()(src, dst, alpha_s, alpha_d,
                                               h_all, maxa, tgt)
    emb = _temporal_call(num0.reshape(T, TGT, D), num1.reshape(T, TGT, D),
                         den0.reshape(T, TGT, 16), den1.reshape(T, TGT, 16),
                         pos_emb, Wq, Wk, Wv, RH, RS)
    return _cls_call(emb, Wc1, bc1, Wc2, bc2)


# speedup vs baseline: 2.7096x; 1.1272x over previous
import functools

import jax
import jax.numpy as jnp
from jax import lax
from jax.experimental import pallas as pl
from jax.experimental.pallas import tpu as pltpu
from jax.experimental.pallas import tpu_sc as plsc

N = 10000
T = 8
F = 128
E = 320000
D = 128
H = 8
DH = 16

NC = 2
NS = 16
EDGES_PER_TILE = E // (NC * NS)
C = 160
CC = 400
TGT = 2048
TPT = TGT // NS
GARB = TGT
SLOTS = TGT + 16
SROWS = SLOTS // NS
CBUF = EDGES_PER_TILE + 2 * C + 16

BLK1 = 2000
BLK3 = 256


def _prep_body(x_ref, w_ref, a_ref, h_ref, as_ref, ad_ref, maxa_ref):
    b = pl.program_id(0)
    hb = jnp.dot(x_ref[...], w_ref[...], preferred_element_type=jnp.float32)
    ab = jnp.dot(hb, a_ref[...], preferred_element_type=jnp.float32)
    h_ref[...] = hb
    as_ref[...] = ab[:, 0:16]
    ad_ref[...] = ab[:, 16:32]
    lane = lax.broadcasted_iota(jnp.int32, (1, 128), 1)
    colmax = jnp.max(ab, axis=0, keepdims=True)
    m = jnp.where(lane < 16, colmax, 0.0)

    @pl.when(b == 0)
    def _():
        maxa_ref[...] = m

    @pl.when(b != 0)
    def _():
        maxa_ref[...] = jnp.maximum(maxa_ref[...], m)


def _prep_call(x2, W_s, A128):
    nblk = (T * N) // BLK1
    return pl.pallas_call(
        _prep_body,
        grid=(nblk,),
        in_specs=[
            pl.BlockSpec((BLK1, F), lambda b: (b, 0)),
            pl.BlockSpec((F, D), lambda b: (0, 0)),
            pl.BlockSpec((D, 128), lambda b: (0, 0)),
        ],
        out_specs=[
            pl.BlockSpec((BLK1, D), lambda b: (b, 0)),
            pl.BlockSpec((BLK1, 16), lambda b: (b, 0)),
            pl.BlockSpec((BLK1, 16), lambda b: (b, 0)),
            pl.BlockSpec((1, 128), lambda b: (0, 0)),
        ],
        out_shape=[
            jax.ShapeDtypeStruct((T * N, D), jnp.float32),
            jax.ShapeDtypeStruct((T * N, 16), jnp.float32),
            jax.ShapeDtypeStruct((T * N, 16), jnp.float32),
            jax.ShapeDtypeStruct((1, 128), jnp.float32),
        ],
    )(x2, W_s, A128)


def _edge_body(src_hbm, dst_hbm, as_hbm, ad_hbm, h_hbm, maxa_hbm, tgt_hbm,
               num0, num1, den0, den1,
               acc_num, acc_den,
               smap, tgt_all, csrc, cdst, cslot, in_s, in_d,
               idx_so_a, idx_do_a, slot_a, arow_s_a, arow_d_a, hrow_a, exb_a,
               m_buf, znum, zden, rep, gnum, gden, sem_a):
    sem_e = sem_a
    cid = lax.axis_index("c")
    sid = lax.axis_index("s")
    tile_base = (cid * NS + sid) * EDGES_PER_TILE

    sets = ((idx_so_a, idx_do_a, slot_a, arow_s_a, arow_d_a, hrow_a, exb_a,
             sem_a),)

    pltpu.sync_copy(maxa_hbm, m_buf)
    mvec = m_buf[0, pl.ds(0, 16)]

    pltpu.sync_copy(tgt_hbm, tgt_all)

    garb16 = jnp.full((16,), GARB, jnp.int32)
    z16i = jnp.zeros((16,), jnp.int32)
    z16 = jnp.zeros((16,), jnp.float32)
    lane16 = lax.iota(jnp.int32, 16)

    def zrow(i, _):
        for j in range(D // 16):
            znum[i, pl.ds(16 * j, 16)] = z16
        zden[i, pl.ds(0, 16)] = z16
        return 0

    lax.fori_loop(0, SROWS, zrow, 0)

    def fillg(i, _):
        smap[pl.ds(16 * i, 16)] = garb16
        return 0

    lax.fori_loop(0, N // 16, fillg, 0)

    def scat(i, _):
        tvec = tgt_all[pl.ds(16 * i, 16)]
        plsc.store_scatter(smap, [tvec], lane16 + 16 * i)
        return 0

    lax.fori_loop(0, TGT // 16, scat, 0)

    for i in range(TPT // 16):
        tvec = tgt_all[pl.ds(sid * TPT + 16 * i, 16)]
        rep[pl.ds(16 * i, 16)] = plsc.load_gather(smap, [tvec])

    def comp_chunk(k, wpos):
        pltpu.sync_copy(src_hbm.at[pl.ds(tile_base + k * CC, CC)], in_s)
        pltpu.sync_copy(dst_hbm.at[pl.ds(tile_base + k * CC, CC)], in_d)

        def inner(i, w):
            svec = in_s[pl.ds(16 * i, 16)]
            dvec = in_d[pl.ds(16 * i, 16)]
            sl = plsc.load_gather(smap, [dvec])
            mask = sl < GARB
            plsc.store_compressed(csrc.at[pl.ds(w, 16)], svec, mask=mask)
            plsc.store_compressed(cdst.at[pl.ds(w, 16)], dvec, mask=mask)
            plsc.store_compressed(cslot.at[pl.ds(w, 16)], sl, mask=mask)
            cnt = plsc.all_reduce_population_count(mask)
            return w + cnt[0]

        return lax.fori_loop(0, CC // 16, inner, wpos)

    wpos = lax.fori_loop(0, EDGES_PER_TILE // CC, comp_chunk, 0)

    nch = (wpos + C - 1) // C
    npad16 = (nch * C - wpos + 15) // 16

    def padf(j, _):
        base = wpos + 16 * j
        csrc[pl.ds(base, 16)] = z16i
        cdst[pl.ds(base, 16)] = z16i
        cslot[pl.ds(base, 16)] = garb16
        return 0

    lax.fori_loop(0, npad16, padf, 0)

    def t_body(t, _):
        off = sid * SROWS
        pltpu.sync_copy(znum, acc_num.at[pl.ds(off, SROWS), :])
        pltpu.sync_copy(zden, acc_den.at[pl.ds(off, SROWS), :])
        plsc.subcore_barrier()

        tN = t * N

        def fire(k, s):
            idx_so, idx_do, slot_c, arow_s, arow_d, hrow, exb, sem = sets[s]
            base = k * C

            def offs(i, _):
                idx_so[pl.ds(16 * i, 16)] = csrc[pl.ds(base + 16 * i, 16)] + tN
                idx_do[pl.ds(16 * i, 16)] = cdst[pl.ds(base + 16 * i, 16)] + tN
                slot_c[pl.ds(16 * i, 16)] = cslot[pl.ds(base + 16 * i, 16)]
                return 0

            lax.fori_loop(0, C // 16, offs, 0)
            pltpu.async_copy(as_hbm.at[idx_so], arow_s, sem)
            pltpu.async_copy(ad_hbm.at[idx_do], arow_d, sem)
            pltpu.async_copy(h_hbm.at[idx_so], hrow, sem)

        def finish(s):
            idx_so, idx_do, slot_c, arow_s, arow_d, hrow, exb, sem = sets[s]
            pltpu.make_async_copy(as_hbm.at[idx_so], arow_s, sem).wait()
            pltpu.make_async_copy(ad_hbm.at[idx_do], arow_d, sem).wait()
            pltpu.make_async_copy(h_hbm.at[idx_so], hrow, sem).wait()

            def edge_body(i, _):
                sv = arow_s[i, :]
                dv = arow_d[i, :]
                z = sv + dv
                lr = jnp.maximum(z, 0.2 * z)
                zm = mvec + dv
                lrm = jnp.maximum(zm, 0.2 * zm)
                ex = jnp.exp(lr - lrm)
                exb[i, :] = ex
                for hh in range(H):
                    sc = ex[hh]
                    hv = hrow[i, pl.ds(16 * hh, 16)]
                    hrow[i, pl.ds(16 * hh, 16)] = sc * hv
                return 0

            lax.fori_loop(0, C, edge_body, 0)

            pltpu.sync_copy(hrow, acc_num.at[slot_c], add=True)
            pltpu.sync_copy(exb, acc_den.at[slot_c], add=True)

        def body1(k, _):
            fire(k, 0)
            finish(0)
            return 0

        lax.fori_loop(0, nch, body1, 0)
        plsc.subcore_barrier()

        cpn = pltpu.async_copy(acc_num.at[rep], gnum, sem_e)
        cpd = pltpu.async_copy(acc_den.at[rep], gden, sem_e)
        cpn.wait()
        cpd.wait()
        obase = t * TGT + sid * TPT

        @pl.when(cid == 0)
        def _():
            pltpu.sync_copy(gnum, num0.at[pl.ds(obase, TPT), :])
            pltpu.sync_copy(gden, den0.at[pl.ds(obase, TPT), :])

        @pl.when(cid == 1)
        def _():
            pltpu.sync_copy(gnum, num1.at[pl.ds(obase, TPT), :])
            pltpu.sync_copy(gden, den1.at[pl.ds(obase, TPT), :])

        plsc.subcore_barrier()
        return 0

    lax.fori_loop(0, T, t_body, 0)


@functools.cache
def _make_edge_call():
  return pl.kernel(
    _edge_body,
    out_type=[
        jax.ShapeDtypeStruct((T * TGT, D), jnp.float32),
        jax.ShapeDtypeStruct((T * TGT, D), jnp.float32),
        jax.ShapeDtypeStruct((T * TGT, 16), jnp.float32),
        jax.ShapeDtypeStruct((T * TGT, 16), jnp.float32),
    ],
    mesh=plsc.VectorSubcoreMesh(core_axis_name="c", subcore_axis_name="s",
                                num_cores=NC, num_subcores=NS),
    scratch_types=[
        pltpu.VMEM_SHARED((SLOTS, D), jnp.float32),
        pltpu.VMEM_SHARED((SLOTS, 16), jnp.float32),
        pltpu.VMEM((N,), jnp.int32),
        pltpu.VMEM((TGT,), jnp.int32),
        pltpu.VMEM((CBUF,), jnp.int32),
        pltpu.VMEM((CBUF,), jnp.int32),
        pltpu.VMEM((CBUF,), jnp.int32),
        pltpu.VMEM((CC,), jnp.int32),
        pltpu.VMEM((CC,), jnp.int32),
        pltpu.VMEM((C,), jnp.int32),
        pltpu.VMEM((C,), jnp.int32),
        pltpu.VMEM((C,), jnp.int32),
        pltpu.VMEM((C, 16), jnp.float32),
        pltpu.VMEM((C, 16), jnp.float32),
        pltpu.VMEM((C, D), jnp.float32),
        pltpu.VMEM((C, 16), jnp.float32),
        pltpu.VMEM((1, 128), jnp.float32),
        pltpu.VMEM((SROWS, D), jnp.float32),
        pltpu.VMEM((SROWS, 16), jnp.float32),
        pltpu.VMEM((TPT,), jnp.int32),
        pltpu.VMEM((TPT, D), jnp.float32),
        pltpu.VMEM((TPT, 16), jnp.float32),
        pltpu.SemaphoreType.DMA,
    ],
    compiler_params=pltpu.CompilerParams(use_tc_tiling_on_sc=False,
                                         needs_layout_passes=False),
  )


def _temporal_body(n0, n1, d0, d1, pos, wq, wk, wv, rh, rs, out):
    num2 = (n0[...] + n1[...]).reshape(T * BLK3, D)
    den2 = (d0[...] + d1[...]).reshape(T * BLK3, 16)
    den128 = jnp.dot(den2, rh[...], preferred_element_type=jnp.float32) + 1e-16
    g = num2 / den128
    g = jnp.where(g > 0, g, jnp.exp(g) - 1.0)
    s3 = g.reshape(T, BLK3, D) + pos[...][:, None, :]
    s2 = s3.reshape(T * BLK3, D)
    k2 = jnp.dot(s2, wk[...], preferred_element_type=jnp.float32).reshape(T, BLK3, D)
    v2 = jnp.dot(s2, wv[...], preferred_element_type=jnp.float32).reshape(T, BLK3, D)
    q7 = jnp.dot(s3[T - 1], wq[...], preferred_element_type=jnp.float32)
    scs = [jnp.dot(q7 * k2[t], rs[...], preferred_element_type=jnp.float32) * 0.25
           for t in range(T)]
    m = scs[0]
    for t in range(1, T):
        m = jnp.maximum(m, scs[t])
    exs = [jnp.exp(s - m) for s in scs]
    dsum = exs[0]
    for t in range(1, T):
        dsum = dsum + exs[t]
    rh8 = rh[...][0:8, :]
    tout = jnp.zeros((BLK3, D), jnp.float32)
    for t in range(T):
        w_t = exs[t] / dsum
        tout = tout + jnp.dot(w_t, rh8, preferred_element_type=jnp.float32) * v2[t]
    out[...] = tout + s3[T - 1]


def _temporal_call(num0, num1, den0, den1, pos_emb, Wq, Wk, Wv, RH, RS):
    nblk = TGT // BLK3
    return pl.pallas_call(
        _temporal_body,
        grid=(nblk,),
        in_specs=[
            pl.BlockSpec((T, BLK3, D), lambda b: (0, b, 0)),
            pl.BlockSpec((T, BLK3, D), lambda b: (0, b, 0)),
            pl.BlockSpec((T, BLK3, 16), lambda b: (0, b, 0)),
            pl.BlockSpec((T, BLK3, 16), lambda b: (0, b, 0)),
            pl.BlockSpec((T, D), lambda b: (0, 0)),
            pl.BlockSpec((D, D), lambda b: (0, 0)),
            pl.BlockSpec((D, D), lambda b: (0, 0)),
            pl.BlockSpec((D, D), lambda b: (0, 0)),
            pl.BlockSpec((16, D), lambda b: (0, 0)),
            pl.BlockSpec((D, 8), lambda b: (0, 0)),
        ],
        out_specs=pl.BlockSpec((BLK3, D), lambda b: (b, 0)),
        out_shape=jax.ShapeDtypeStruct((TGT, D), jnp.float32),
    )(num0, num1, den0, den1, pos_emb, Wq, Wk, Wv, RH, RS)


def _cls_body(e_ref, w1, b1, w2, b2, out):
    e = e_ref[...]
    p = e[0:1024] * e[1024:2048]
    h1 = jnp.dot(p, w1[...], preferred_element_type=jnp.float32) + b1[...]
    h1 = jnp.maximum(h1, 0.0)
    out[...] = jnp.dot(h1, w2[...], preferred_element_type=jnp.float32) + b2[...]


def _cls_call(emb, Wc1, bc1, Wc2, bc2):
    return pl.pallas_call(
        _cls_body,
        out_shape=jax.ShapeDtypeStruct((TGT // 2, 2), jnp.float32),
    )(emb, Wc1, bc1.reshape(1, D), Wc2, bc2.reshape(1, 2))


def kernel(graphs_x, edge_index, nids, W_s, a_src, a_dst, Wq, Wk, Wv,
           pos_emb, Wc1, bc1, Wc2, bc2):
    x2 = graphs_x.reshape(T * N, F)
    src = edge_index[0].astype(jnp.int32)
    dst = edge_index[1].astype(jnp.int32)
    tgt = jnp.concatenate([nids[:, 0], nids[:, 1]]).astype(jnp.int32)

    head_of_lane = jnp.arange(D) // DH
    blockdiag = (head_of_lane[:, None] == jnp.arange(H)[None, :]).astype(jnp.float32)
    Asrc = blockdiag * a_src.reshape(D)[:, None]
    Adst = blockdiag * a_dst.reshape(D)[:, None]
    A128 = jnp.concatenate(
        [Asrc, Asrc, Adst, Adst, jnp.zeros((D, 96), jnp.float32)], axis=1)
    RH = (jnp.arange(16)[:, None] == head_of_lane[None, :]).astype(jnp.float32)
    RS = RH[:8].T

    h_all, alpha_s, alpha_d, maxa = _prep_call(x2, W_s, A128)
    num0, num1, den0, den1 = _make_edge_call()(src, dst, alpha_s, alpha_d,
                                               h_all, maxa, tgt)
    emb = _temporal_call(num0.reshape(T, TGT, D), num1.reshape(T, TGT, D),
                         den0.reshape(T, TGT, 16), den1.reshape(T, TGT, 16),
                         pos_emb, Wq, Wk, Wv, RH, RS)
    return _cls_call(emb, Wc1, bc1, Wc2, bc2)
